# Initial kernel scaffold; baseline (speedup 1.0000x reference)
import jax
import jax.numpy as jnp
from jax.experimental import pallas as pl


def kernel(nodes, history_uv, history_r, v2e_w, u2e_w, r2e_w, w_r1_W, w_r1_b, w_r2_W, w_r2_b, att1_W, att1_b, att2_W, att2_b, att3_W, att3_b):
    raise NotImplementedError("write your pallas kernel here")


# speedup vs baseline: 2.4764x; 2.4764x over previous
import functools

import jax
import jax.numpy as jnp
from jax import lax
from jax.experimental import pallas as pl
from jax.experimental.pallas import tpu as pltpu
from jax.experimental.pallas import tpu_sc as plsc

D = 64
L = 50
B = 1024
BL = B * L
NW = 32
ROWS_W = BL // NW
CHUNK = 80
NCHUNK = ROWS_W // CHUNK
UROWS_W = B // NW

BN = 16
RB = BN * L


def _sc_gather(v2e_w, u2e_w, idx2d, nodes):
  mesh = plsc.VectorSubcoreMesh(core_axis_name="c", subcore_axis_name="s")

  @functools.partial(
      pl.kernel,
      mesh=mesh,
      compiler_params=pltpu.CompilerParams(use_tc_tiling_on_sc=False),
      out_type=(
          jax.ShapeDtypeStruct((BL, D), jnp.float32),
          jax.ShapeDtypeStruct((B, D), jnp.float32),
      ),
      scratch_types=[
          pltpu.VMEM((NCHUNK, CHUNK), jnp.int32),
          pltpu.VMEM((ROWS_W, D), jnp.float32),
          pltpu.VMEM((UROWS_W,), jnp.int32),
          pltpu.VMEM((UROWS_W, D), jnp.float32),
          pltpu.SemaphoreType.DMA,
          pltpu.SemaphoreType.DMA,
      ],
  )
  def k(v2e_hbm, u2e_hbm, idx_hbm, nodes_hbm, euv_out, uv_out,
        idx_v, rows_v, nidx_v, urows_v, sem, usem):
    wid = lax.axis_index("s") * 2 + lax.axis_index("c")

    pltpu.sync_copy(idx_hbm.at[wid], idx_v)
    copies = []
    for j in range(NCHUNK):
      copies.append(pltpu.async_copy(
          v2e_hbm.at[idx_v.at[j]], rows_v.at[pl.ds(j * CHUNK, CHUNK)], sem))
    pltpu.sync_copy(nodes_hbm.at[pl.ds(wid * UROWS_W, UROWS_W)], nidx_v)
    ucopy = pltpu.async_copy(u2e_hbm.at[nidx_v], urows_v, usem)
    for c in copies:
      c.wait()
    pltpu.sync_copy(rows_v, euv_out.at[pl.ds(wid * ROWS_W, ROWS_W)])
    ucopy.wait()
    pltpu.sync_copy(urows_v, uv_out.at[pl.ds(wid * UROWS_W, UROWS_W)])

  return k(v2e_w, u2e_w, idx2d, nodes)


def _tc_body(euv_ref, r_ref, uv_ref, r2e_ref,
             w1a_ref, w1b_ref, w2_ref, a1a_ref, a1b_ref, a2_ref, a3_ref,
             b1_ref, b2_ref, ba1_ref, ba2_ref, out_ref):
  f32 = jnp.float32
  eu = euv_ref[...]
  r = r_ref[...]
  uv = uv_ref[...]

  r2e_proj = jnp.dot(r2e_ref[...], w1b_ref[...],
                     preferred_element_type=f32)
  rsel = jnp.zeros((RB, D), f32)
  for k in range(5):
    rsel = rsel + jnp.where(r == k, 1.0, 0.0) * r2e_proj[k:k + 1, :]

  h = jax.nn.relu(jnp.dot(eu, w1a_ref[...], preferred_element_type=f32)
                  + rsel + b1_ref[...])
  o = jax.nn.relu(jnp.dot(h, w2_ref[...], preferred_element_type=f32)
                  + b2_ref[...])

  g = jnp.dot(uv, a1b_ref[...], preferred_element_type=f32)
  row_ids = lax.broadcasted_iota(jnp.int32, (RB, BN), 0) // L
  node_ids = lax.broadcasted_iota(jnp.int32, (RB, BN), 1)
  pt = jnp.where(row_ids == node_ids, 1.0, 0.0)
  t = jax.nn.relu(jnp.dot(o, a1a_ref[...], preferred_element_type=f32)
                  + jnp.dot(pt, g, preferred_element_type=f32) + ba1_ref[...])
  a2 = jax.nn.relu(jnp.dot(t, a2_ref[...], preferred_element_type=f32)
                   + ba2_ref[...])

  lg = jnp.sum(a2 * a3_ref[...], axis=1, keepdims=True)
  e = jnp.exp(lg - jnp.max(lg))
  st = jnp.concatenate([e * o, jnp.broadcast_to(e, (RB, D))], axis=1)
  seg = jnp.dot(pt.T, st, preferred_element_type=f32)
  out_ref[...] = seg[:, :D] / seg[:, D:]


def _tc_aggregate(euv_flat, r_col, uv_rep, r2e_pad, w1a_t, w1b_t, w2_t,
                  a1a_t, a1b_t, a2_t, a3, b1, b2, ba1, ba2, interpret=False):
  grid = B // BN
  blk = lambda shape: pl.BlockSpec(shape, lambda i: (0, 0))
  return pl.pallas_call(
      _tc_body,
      grid=(grid,),
      in_specs=[
          pl.BlockSpec((RB, D), lambda i: (i, 0)),
          pl.BlockSpec((RB, 1), lambda i: (i, 0)),
          pl.BlockSpec((BN, D), lambda i: (i, 0)),
          blk((8, D)),
          blk((D, D)), blk((D, D)), blk((D, D)),
          blk((D, D)), blk((D, D)), blk((D, D)), blk((1, D)),
          blk((1, D)), blk((1, D)), blk((1, D)), blk((1, D)),
      ],
      out_specs=pl.BlockSpec((BN, D), lambda i: (i, 0)),
      out_shape=jax.ShapeDtypeStruct((B, D), jnp.float32),
      interpret=interpret,
  )(euv_flat, r_col, uv_rep, r2e_pad, w1a_t, w1b_t, w2_t,
    a1a_t, a1b_t, a2_t, a3, b1, b2, ba1, ba2)


def kernel(nodes, history_uv, history_r, v2e_w, u2e_w, r2e_w,
           w_r1_W, w_r1_b, w_r2_W, w_r2_b,
           att1_W, att1_b, att2_W, att2_b, att3_W, att3_b):
  idx2d = history_uv.reshape(NW, NCHUNK, CHUNK)
  euv_flat, uv_rep = _sc_gather(v2e_w, u2e_w, idx2d, nodes)

  r_col = history_r.reshape(BL, 1)
  r2e_pad = jnp.zeros((8, D), jnp.float32).at[:5].set(r2e_w)
  w1a_t = w_r1_W[:, :D].T
  w1b_t = w_r1_W[:, D:].T
  w2_t = w_r2_W.T
  a1a_t = att1_W[:, :D].T
  a1b_t = att1_W[:, D:].T
  a2_t = att2_W.T
  a3 = att3_W.reshape(1, D)
  b1 = w_r1_b.reshape(1, D)
  b2 = w_r2_b.reshape(1, D)
  ba1 = att1_b.reshape(1, D)
  ba2 = att2_b.reshape(1, D)

  return _tc_aggregate(euv_flat, r_col, uv_rep, r2e_pad, w1a_t, w1b_t, w2_t,
                       a1a_t, a1b_t, a2_t, a3, b1, b2, ba1, ba2)


# speedup vs baseline: 3.0299x; 1.2235x over previous
import functools

import jax
import jax.numpy as jnp
from jax import lax
from jax.experimental import pallas as pl
from jax.experimental.pallas import tpu as pltpu
from jax.experimental.pallas import tpu_sc as plsc

D = 64
L = 50
B = 1024
BL = B * L
NW = 32
ROWS_W = BL // NW
CHUNK = 80
NCHUNK = ROWS_W // CHUNK
UROWS_W = B // NW

BN = 32
RB2 = BN * (L // 2)


def _sc_gather(v2e_w, u2e_w, idx2d, nodes):
  mesh = plsc.VectorSubcoreMesh(core_axis_name="c", subcore_axis_name="s")

  @functools.partial(
      pl.kernel,
      mesh=mesh,
      compiler_params=pltpu.CompilerParams(use_tc_tiling_on_sc=False),
      out_type=(
          jax.ShapeDtypeStruct((BL, D), jnp.float32),
          jax.ShapeDtypeStruct((B, D), jnp.float32),
      ),
      scratch_types=[
          pltpu.VMEM((NCHUNK, CHUNK), jnp.int32),
          pltpu.VMEM((ROWS_W, D), jnp.float32),
          pltpu.VMEM((UROWS_W,), jnp.int32),
          pltpu.VMEM((UROWS_W, D), jnp.float32),
          pltpu.SemaphoreType.DMA,
          pltpu.SemaphoreType.DMA,
      ],
  )
  def k(v2e_hbm, u2e_hbm, idx_hbm, nodes_hbm, euv_out, uv_out,
        idx_v, rows_v, nidx_v, urows_v, sem, usem):
    wid = lax.axis_index("s") * 2 + lax.axis_index("c")

    pltpu.sync_copy(idx_hbm.at[wid], idx_v)
    copies = []
    for j in range(NCHUNK):
      copies.append(pltpu.async_copy(
          v2e_hbm.at[idx_v.at[j]], rows_v.at[pl.ds(j * CHUNK, CHUNK)], sem))
    pltpu.sync_copy(nodes_hbm.at[pl.ds(wid * UROWS_W, UROWS_W)], nidx_v)
    ucopy = pltpu.async_copy(u2e_hbm.at[nidx_v], urows_v, usem)
    for c in copies:
      c.wait()
    pltpu.sync_copy(rows_v, euv_out.at[pl.ds(wid * ROWS_W, ROWS_W)])
    ucopy.wait()
    pltpu.sync_copy(urows_v, uv_out.at[pl.ds(wid * UROWS_W, UROWS_W)])

  return k(v2e_w, u2e_w, idx2d, nodes)


def _mm(a, b):
  return jnp.dot(a, b, preferred_element_type=jnp.float32)


def _mm_tl(a, b):
  return lax.dot_general(a, b, (((0,), (0,)), ((), ())),
                         preferred_element_type=jnp.float32)


def _tc_body(euv_ref, rt_ref, uv_ref, r2e_ref,
             w1a_ref, w1b_ref, w2_ref, a1a_ref, a1b_ref, a2_ref, a3_ref,
             b1_ref, b2_ref, ba1_ref, ba2_ref, out_ref):
  f32 = jnp.float32
  eu2 = euv_ref[...]
  eL, eR = eu2[:, :D], eu2[:, D:]
  rt = rt_ref[0]
  uv = uv_ref[...]

  proj = _mm(r2e_ref[...], w1b_ref[...])
  ohL = jnp.concatenate(
      [jnp.where(rt[0:1, :] == k, 1.0, 0.0) for k in range(8)], axis=0)
  ohR = jnp.concatenate(
      [jnp.where(rt[1:2, :] == k, 1.0, 0.0) for k in range(8)], axis=0)
  rselL = _mm_tl(ohL, proj)
  rselR = _mm_tl(ohR, proj)

  b1, b2, ba1, ba2 = b1_ref[...], b2_ref[...], ba1_ref[...], ba2_ref[...]
  hL = jax.nn.relu(_mm(eL, w1a_ref[...]) + rselL + b1)
  hR = jax.nn.relu(_mm(eR, w1a_ref[...]) + rselR + b1)
  oL = jax.nn.relu(_mm(hL, w2_ref[...]) + b2)
  oR = jax.nn.relu(_mm(hR, w2_ref[...]) + b2)

  g = _mm(uv, a1b_ref[...])
  pair_ids = lax.broadcasted_iota(jnp.int32, (RB2, BN), 0) // (L // 2)
  node_ids = lax.broadcasted_iota(jnp.int32, (RB2, BN), 1)
  pt = jnp.where(pair_ids == node_ids, 1.0, 0.0)
  q = _mm(pt, g) + ba1
  tL = jax.nn.relu(_mm(oL, a1a_ref[...]) + q)
  tR = jax.nn.relu(_mm(oR, a1a_ref[...]) + q)
  aL = jax.nn.relu(_mm(tL, a2_ref[...]) + ba2)
  aR = jax.nn.relu(_mm(tR, a2_ref[...]) + ba2)

  a3 = a3_ref[...]
  lgL = jnp.sum(aL * a3, axis=1, keepdims=True)
  lgR = jnp.sum(aR * a3, axis=1, keepdims=True)
  m = jnp.maximum(jnp.max(lgL), jnp.max(lgR))
  expL = jnp.exp(lgL - m)
  expR = jnp.exp(lgR - m)
  wsum = expL * oL + expR * oR
  dcol = jnp.broadcast_to(expL + expR, (RB2, D))
  st = jnp.concatenate([wsum, dcol], axis=1)
  seg_t = _mm_tl(st, pt)
  out_ref[0] = seg_t[:D, :] / seg_t[D:, :]


def _tc_aggregate(euv128, rt, uv_rep, r2e_pad, w1a_t, w1b_t, w2_t,
                  a1a_t, a1b_t, a2_t, a3, b1, b2, ba1, ba2, interpret=False):
  grid = B // BN
  blk = lambda shape: pl.BlockSpec(shape, lambda i: (0, 0))
  return pl.pallas_call(
      _tc_body,
      grid=(grid,),
      in_specs=[
          pl.BlockSpec((RB2, 2 * D), lambda i: (i, 0)),
          pl.BlockSpec((1, 8, RB2), lambda i: (i, 0, 0)),
          pl.BlockSpec((BN, D), lambda i: (i, 0)),
          blk((8, D)),
          blk((D, D)), blk((D, D)), blk((D, D)),
          blk((D, D)), blk((D, D)), blk((D, D)), blk((1, D)),
          blk((1, D)), blk((1, D)), blk((1, D)), blk((1, D)),
      ],
      out_specs=pl.BlockSpec((1, D, BN), lambda i: (i, 0, 0)),
      out_shape=jax.ShapeDtypeStruct((B // BN, D, BN), jnp.float32),
      interpret=interpret,
  )(euv128, rt, uv_rep, r2e_pad, w1a_t, w1b_t, w2_t,
    a1a_t, a1b_t, a2_t, a3, b1, b2, ba1, ba2)


def kernel(nodes, history_uv, history_r, v2e_w, u2e_w, r2e_w,
           w_r1_W, w_r1_b, w_r2_W, w_r2_b,
           att1_W, att1_b, att2_W, att2_b, att3_W, att3_b):
  idx2d = history_uv.reshape(NW, NCHUNK, CHUNK)
  euv_flat, uv_rep = _sc_gather(v2e_w, u2e_w, idx2d, nodes)

  euv128 = euv_flat.reshape(BL // 2, 2 * D)
  rt = jnp.pad(
      jnp.transpose(history_r.reshape(B // BN, RB2, 2), (0, 2, 1)),
      ((0, 0), (0, 6), (0, 0))).astype(jnp.float32)
  r2e_pad = jnp.zeros((8, D), jnp.float32).at[:5].set(r2e_w)
  w1a_t = w_r1_W[:, :D].T
  w1b_t = w_r1_W[:, D:].T
  w2_t = w_r2_W.T
  a1a_t = att1_W[:, :D].T
  a1b_t = att1_W[:, D:].T
  a2_t = att2_W.T
  a3 = att3_W.reshape(1, D)
  b1 = w_r1_b.reshape(1, D)
  b2 = w_r2_b.reshape(1, D)
  ba1 = att1_b.reshape(1, D)
  ba2 = att2_b.reshape(1, D)

  out3 = _tc_aggregate(euv128, rt, uv_rep, r2e_pad, w1a_t, w1b_t, w2_t,
                       a1a_t, a1b_t, a2_t, a3, b1, b2, ba1, ba2)
  return jnp.transpose(out3, (0, 2, 1)).reshape(B, D)


# speedup vs baseline: 3.1449x; 1.0380x over previous
import functools

import jax
import jax.numpy as jnp
from jax import lax
from jax.experimental import pallas as pl
from jax.experimental.pallas import tpu as pltpu
from jax.experimental.pallas import tpu_sc as plsc

D = 64
L = 50
B = 1024
BL = B * L
NW = 32
ROWS_W = BL // NW
CHUNK = 80
NCHUNK = ROWS_W // CHUNK
UROWS_W = B // NW

BN = 64
RB2 = BN * (L // 2)


def _sc_gather(v2e_w, u2e_w, idx2d, nodes):
  mesh = plsc.VectorSubcoreMesh(core_axis_name="c", subcore_axis_name="s")

  @functools.partial(
      pl.kernel,
      mesh=mesh,
      compiler_params=pltpu.CompilerParams(use_tc_tiling_on_sc=False),
      out_type=(
          jax.ShapeDtypeStruct((BL, D), jnp.float32),
          jax.ShapeDtypeStruct((B, D), jnp.float32),
      ),
      scratch_types=[
          pltpu.VMEM((NCHUNK, CHUNK), jnp.int32),
          pltpu.VMEM((ROWS_W, D), jnp.float32),
          pltpu.VMEM((UROWS_W,), jnp.int32),
          pltpu.VMEM((UROWS_W, D), jnp.float32),
          pltpu.SemaphoreType.DMA((NCHUNK,)),
          pltpu.SemaphoreType.DMA,
          pltpu.SemaphoreType.DMA,
      ],
  )
  def k(v2e_hbm, u2e_hbm, idx_hbm, nodes_hbm, euv_out, uv_out,
        idx_v, rows_v, nidx_v, urows_v, sems, wsem, usem):
    wid = lax.axis_index("s") * 2 + lax.axis_index("c")

    pltpu.sync_copy(idx_hbm.at[wid], idx_v)
    copies = []
    for j in range(NCHUNK):
      copies.append(pltpu.async_copy(
          v2e_hbm.at[idx_v.at[j]],
          rows_v.at[pl.ds(j * CHUNK, CHUNK)], sems.at[j]))
    pltpu.sync_copy(nodes_hbm.at[pl.ds(wid * UROWS_W, UROWS_W)], nidx_v)
    ucopy = pltpu.async_copy(u2e_hbm.at[nidx_v], urows_v, usem)
    wcopies = []
    for j in range(NCHUNK):
      copies[j].wait()
      wcopies.append(pltpu.async_copy(
          rows_v.at[pl.ds(j * CHUNK, CHUNK)],
          euv_out.at[pl.ds(wid * ROWS_W + j * CHUNK, CHUNK)], wsem))
    ucopy.wait()
    pltpu.sync_copy(urows_v, uv_out.at[pl.ds(wid * UROWS_W, UROWS_W)])
    for w in wcopies:
      w.wait()

  return k(v2e_w, u2e_w, idx2d, nodes)


def _mm(a, b):
  return jnp.dot(a, b, preferred_element_type=jnp.float32)


def _mm_tl(a, b):
  return lax.dot_general(a, b, (((0,), (0,)), ((), ())),
                         preferred_element_type=jnp.float32)


def _tc_body(euv_ref, rt_ref, uv_ref, r2e_ref, w1b_ref,
             w1bd_ref, w2bd_ref, a1bd_ref, a2bd_ref, a1b_ref,
             a3l_ref, a3r_ref, bias_ref,
             pt_ref, ptt_ref, out_ref):
  eu2 = euv_ref[...]
  rt = rt_ref[0]
  uv = uv_ref[...]

  proj = _mm(r2e_ref[...], w1b_ref[...])
  ohL = jnp.concatenate(
      [jnp.where(rt[0:1, :] == k, 1.0, 0.0) for k in range(8)], axis=0)
  ohR = jnp.concatenate(
      [jnp.where(rt[1:2, :] == k, 1.0, 0.0) for k in range(8)], axis=0)
  rsel2 = jnp.concatenate([_mm_tl(ohL, proj), _mm_tl(ohR, proj)], axis=1)

  bias = bias_ref[...]
  h2 = jax.nn.relu(_mm(eu2, w1bd_ref[...]) + rsel2 + bias[0:1, :])
  o2 = jax.nn.relu(_mm(h2, w2bd_ref[...]) + bias[1:2, :])

  g = _mm(uv, a1b_ref[...])
  g2 = jnp.concatenate([g, g], axis=1)
  pt = pt_ref[...]
  q2 = _mm(pt, g2) + bias[2:3, :]
  t2 = jax.nn.relu(_mm(o2, a1bd_ref[...]) + q2)
  a22 = jax.nn.relu(_mm(t2, a2bd_ref[...]) + bias[3:4, :])

  lgL = jnp.sum(a22 * a3l_ref[...], axis=1, keepdims=True)
  lgR = jnp.sum(a22 * a3r_ref[...], axis=1, keepdims=True)
  m = jnp.maximum(jnp.max(lgL), jnp.max(lgR))
  expL = jnp.exp(lgL - m)
  expR = jnp.exp(lgR - m)
  esc = jnp.concatenate([jnp.broadcast_to(expL, (RB2, D)),
                         jnp.broadcast_to(expR, (RB2, D))], axis=1)
  w2s = o2 * esc
  numer = w2s[:, :D] + w2s[:, D:]
  st = jnp.concatenate(
      [numer, jnp.broadcast_to(expL + expR, (RB2, D))], axis=1)
  seg = _mm(ptt_ref[...], st)
  out_ref[0] = seg[:, :D] / seg[:, D:]


def _tc_aggregate(euv128, rt, uv_rep, r2e_pad, w1b_t, w1bd, w2bd, a1bd, a2bd,
                  a1b_t, a3l, a3r, bias, pt, ptt, interpret=False):
  grid = B // BN
  blk = lambda shape: pl.BlockSpec(shape, lambda i: (0, 0))
  return pl.pallas_call(
      _tc_body,
      grid=(grid,),
      in_specs=[
          pl.BlockSpec((RB2, 2 * D), lambda i: (i, 0)),
          pl.BlockSpec((1, 8, RB2), lambda i: (i, 0, 0)),
          pl.BlockSpec((BN, D), lambda i: (i, 0)),
          blk((8, D)), blk((D, D)),
          blk((2 * D, 2 * D)), blk((2 * D, 2 * D)),
          blk((2 * D, 2 * D)), blk((2 * D, 2 * D)),
          blk((D, D)),
          blk((1, 2 * D)), blk((1, 2 * D)), blk((8, 2 * D)),
          blk((RB2, BN)), blk((BN, RB2)),
      ],
      out_specs=pl.BlockSpec((1, BN, D), lambda i: (i, 0, 0)),
      out_shape=jax.ShapeDtypeStruct((B // BN, BN, D), jnp.float32),
      interpret=interpret,
  )(euv128, rt, uv_rep, r2e_pad, w1b_t, w1bd, w2bd, a1bd, a2bd,
    a1b_t, a3l, a3r, bias, pt, ptt)


def kernel(nodes, history_uv, history_r, v2e_w, u2e_w, r2e_w,
           w_r1_W, w_r1_b, w_r2_W, w_r2_b,
           att1_W, att1_b, att2_W, att2_b, att3_W, att3_b):
  idx2d = history_uv.reshape(NW, NCHUNK, CHUNK)
  euv_flat, uv_rep = _sc_gather(v2e_w, u2e_w, idx2d, nodes)

  euv128 = euv_flat.reshape(BL // 2, 2 * D)
  rt = jnp.pad(
      jnp.transpose(history_r.reshape(B // BN, RB2, 2), (0, 2, 1)),
      ((0, 0), (0, 6), (0, 0))).astype(jnp.float32)
  r2e_pad = jnp.zeros((8, D), jnp.float32).at[:5].set(r2e_w)

  def bd(w):
    z = jnp.zeros((D, D), jnp.float32)
    return jnp.block([[w, z], [z, w]])

  w1b_t = w_r1_W[:, D:].T
  w1bd = bd(w_r1_W[:, :D].T)
  w2bd = bd(w_r2_W.T)
  a1bd = bd(att1_W[:, :D].T)
  a2bd = bd(att2_W.T)
  a1b_t = att1_W[:, D:].T
  a3row = att3_W.reshape(1, D)
  zrow = jnp.zeros((1, D), jnp.float32)
  a3l = jnp.concatenate([a3row, zrow], axis=1)
  a3r = jnp.concatenate([zrow, a3row], axis=1)
  bias = jnp.zeros((8, 2 * D), jnp.float32)
  for i, bvec in enumerate((w_r1_b, w_r2_b, att1_b, att2_b)):
    bias = bias.at[i].set(jnp.tile(bvec, 2))
  pair_ids = jnp.arange(RB2)[:, None] // (L // 2)
  pt = (pair_ids == jnp.arange(BN)[None, :]).astype(jnp.float32)
  ptt = pt.T

  out3 = _tc_aggregate(euv128, rt, uv_rep, r2e_pad, w1b_t, w1bd, w2bd,
                       a1bd, a2bd, a1b_t, a3l, a3r, bias, pt, ptt)
  return out3.reshape(B, D)
